# Initial kernel scaffold; baseline (speedup 1.0000x reference)
import jax
import jax.numpy as jnp
from jax.experimental import pallas as pl


def kernel(node_feat, edge_feat, params, edge_index, node_perm_idx, edge_perm_idx, pool_idx):
    raise NotImplementedError("write your pallas kernel here")


# speedup vs baseline: 1.6213x; 1.6213x over previous
import functools

import jax
import jax.numpy as jnp
from jax import lax
from jax.experimental import pallas as pl
from jax.experimental.pallas import tpu as pltpu
from jax.experimental.pallas import tpu_sc as plsc

N = 10000
E = 320000
D = 128
L2 = 16
P = 50000
G = 256
PL = P * L2

NC = 2
NS = 16
NW = NC * NS
EPW = E // NW
KE = 400
NBE = EPW // KE
KEB = 200
NBEB = EPW // KEB
PPW = PL // NW
KP = 200
NBP = PPW // KP
NPAD = 10240
RPS = NPAD // NS

BE = 2560
BP = 400

_mesh = plsc.VectorSubcoreMesh(core_axis_name="c", subcore_axis_name="s")

EPW_A = E // NS
NBE_A = EPW_A // KE
NHALF = NPAD // 2
NQ = NPAD // 4
NPAD4 = 2688
RQ4 = NPAD4 // NS
WQ4 = NQ // NS


def _make_sc_scatter(q0):
    @functools.partial(
        pl.kernel,
        mesh=_mesh,
        out_type=jax.ShapeDtypeStruct((NHALF, D), jnp.float32),
        scratch_types=[
            pltpu.VMEM((KE,), jnp.int32),
            pltpu.VMEM((KE, D), jnp.float32),
            pltpu.VMEM_SHARED((NPAD4, D), jnp.float32),
        ],
        name=f"sc_scatter_q{q0}",
    )
    def _sc_scatter(ef_hbm, dst_hbm, znd_hbm, agg_hbm,
                    idxd, rowbuf, agg_sh):
        c = lax.axis_index("c")
        s = lax.axis_index("s")
        lo = (2 * q0 + c) * NQ
        pltpu.sync_copy(znd_hbm.at[pl.ds(s * RQ4, RQ4)],
                        agg_sh.at[pl.ds(s * RQ4, RQ4)])
        plsc.subcore_barrier()

        def clamp(i, carry):
            sl = pl.ds(i * 16, 16)
            vd = idxd[sl] - lo
            okd = jnp.logical_and(vd >= 0, vd < NQ)
            idxd[sl] = jnp.where(okd, vd, NQ)
            return carry

        def blk(j, carry):
            base = s * EPW_A + j * KE
            pltpu.sync_copy(dst_hbm.at[pl.ds(base, KE)], idxd)
            pltpu.sync_copy(ef_hbm.at[pl.ds(base, KE)], rowbuf)
            lax.fori_loop(0, KE // 16, clamp, 0)
            pltpu.sync_copy(rowbuf, agg_sh.at[idxd], add=True)
            return carry

        lax.fori_loop(0, NBE_A, blk, 0)
        plsc.subcore_barrier()
        pltpu.sync_copy(agg_sh.at[pl.ds(s * WQ4, WQ4)],
                        agg_hbm.at[pl.ds(c * NQ + s * WQ4, WQ4)])

    return _sc_scatter


_sc_scatter_q0 = _make_sc_scatter(0)
_sc_scatter_q1 = _make_sc_scatter(1)


def _make_sc_deg(q0):
    @functools.partial(
        pl.kernel,
        mesh=_mesh,
        out_type=jax.ShapeDtypeStruct((NHALF, D), jnp.float32),
        scratch_types=[
            pltpu.VMEM((KE,), jnp.int32),
            pltpu.VMEM((KE, D), jnp.float32),
            pltpu.VMEM_SHARED((NPAD4, D), jnp.float32),
        ],
        name=f"sc_deg_q{q0}",
    )
    def _sc_deg(src_hbm, znd_hbm, ones_hbm, deg_hbm, idxs, onesbuf, deg_sh):
        c = lax.axis_index("c")
        s = lax.axis_index("s")
        lo = (2 * q0 + c) * NQ
        pltpu.sync_copy(znd_hbm.at[pl.ds(s * RQ4, RQ4)],
                        deg_sh.at[pl.ds(s * RQ4, RQ4)])
        pltpu.sync_copy(ones_hbm, onesbuf)
        plsc.subcore_barrier()

        def clamp(i, carry):
            sl = pl.ds(i * 16, 16)
            vs = idxs[sl] - lo
            oks = jnp.logical_and(vs >= 0, vs < NQ)
            idxs[sl] = jnp.where(oks, vs, NQ)
            return carry

        def blk(j, carry):
            base = s * EPW_A + j * KE
            pltpu.sync_copy(src_hbm.at[pl.ds(base, KE)], idxs)
            lax.fori_loop(0, KE // 16, clamp, 0)
            pltpu.sync_copy(onesbuf, deg_sh.at[idxs], add=True)
            return carry

        lax.fori_loop(0, NBE_A, blk, 0)
        plsc.subcore_barrier()
        pltpu.sync_copy(deg_sh.at[pl.ds(s * WQ4, WQ4)],
                        deg_hbm.at[pl.ds(c * NQ + s * WQ4, WQ4)])

    return _sc_deg


_sc_deg_q0 = _make_sc_deg(0)
_sc_deg_q1 = _make_sc_deg(1)


@functools.partial(
    pl.kernel,
    mesh=_mesh,
    out_type=[
        jax.ShapeDtypeStruct((E, D), jnp.float32),
        jax.ShapeDtypeStruct((E, D), jnp.float32),
        jax.ShapeDtypeStruct((E, D), jnp.float32),
    ],
    scratch_types=[
        pltpu.VMEM((KEB,), jnp.int32),
        pltpu.VMEM((KEB,), jnp.int32),
        pltpu.VMEM((KEB, D), jnp.float32),
        pltpu.VMEM((KEB, D), jnp.float32),
        pltpu.VMEM((KEB, D), jnp.float32),
        pltpu.SemaphoreType.DMA,
    ],
)
def _sc_edge_gather(nfd_hbm, nfs_hbm, stab_hbm, src_hbm, dst_hbm,
                    gd_hbm, gs_hbm, scg_hbm,
                    idxd, idxs, bufd, bufs, bufsc, sem):
    c = lax.axis_index("c")
    s = lax.axis_index("s")
    wid = s * NC + c

    def blk(j, carry):
        base = wid * EPW + j * KEB
        pltpu.sync_copy(dst_hbm.at[pl.ds(base, KEB)], idxd)
        pltpu.sync_copy(src_hbm.at[pl.ds(base, KEB)], idxs)
        pltpu.async_copy(nfd_hbm.at[idxd], bufd, sem).wait()
        pltpu.async_copy(nfs_hbm.at[idxs], bufs, sem).wait()
        pltpu.async_copy(stab_hbm.at[idxd], bufsc, sem).wait()
        pltpu.sync_copy(bufd, gd_hbm.at[pl.ds(base, KEB)])
        pltpu.sync_copy(bufs, gs_hbm.at[pl.ds(base, KEB)])
        pltpu.sync_copy(bufsc, scg_hbm.at[pl.ds(base, KEB)])
        return carry

    lax.fori_loop(0, NBEB, blk, 0)


@functools.partial(
    pl.kernel,
    mesh=_mesh,
    out_type=[
        jax.ShapeDtypeStruct((PL, D), jnp.float32),
        jax.ShapeDtypeStruct((PL, D), jnp.float32),
    ],
    scratch_types=[
        pltpu.VMEM((KP,), jnp.int32),
        pltpu.VMEM((KP,), jnp.int32),
        pltpu.VMEM((KP, D), jnp.float32),
        pltpu.VMEM((KP, D), jnp.float32),
        pltpu.SemaphoreType.DMA,
    ],
)
def _sc_perm_gather(nout_hbm, eout_hbm, npi_hbm, epi_hbm,
                    gn_hbm, ge_hbm,
                    idxn, idxe, bufn, bufe, sem):
    c = lax.axis_index("c")
    s = lax.axis_index("s")
    wid = s * NC + c

    def blk(j, carry):
        base = wid * PPW + j * KP
        pltpu.sync_copy(npi_hbm.at[pl.ds(base, KP)], idxn)
        pltpu.sync_copy(epi_hbm.at[pl.ds(base, KP)], idxe)
        pltpu.async_copy(nout_hbm.at[idxn], bufn, sem).wait()
        pltpu.async_copy(eout_hbm.at[idxe], bufe, sem).wait()
        pltpu.sync_copy(bufn, gn_hbm.at[pl.ds(base, KP)])
        pltpu.sync_copy(bufe, ge_hbm.at[pl.ds(base, KP)])
        return carry

    lax.fori_loop(0, NBP, blk, 0)


def _tc_prep_body(nf, dstw, srcw, deg0, deg1, nfd, nfs, stab):
    x = nf[...]
    nfd[...] = jnp.dot(x, dstw[...], preferred_element_type=jnp.float32)
    nfs[...] = jnp.dot(x, srcw[...], preferred_element_type=jnp.float32)
    deg = jnp.concatenate([deg0[...], deg1[...]], axis=0)[:N, 0:1]
    sc = 2.0 * (1.0 + jnp.log(1.0 + deg) * 1.4426950408889634)
    stab[...] = jnp.broadcast_to(sc, (N, D))


def _tc_prep(nf, dstw, srcw, deg0, deg1):
    return pl.pallas_call(
        _tc_prep_body,
        out_shape=[
            jax.ShapeDtypeStruct((N, D), jnp.float32),
            jax.ShapeDtypeStruct((N, D), jnp.float32),
            jax.ShapeDtypeStruct((N, D), jnp.float32),
        ],
    )(nf, dstw, srcw, deg0, deg1)


def _tc_e1_body(ef, gd, gs, scg, eloopw, srcw, dstw, eb, l1w, l1b,
                h_out, s_out, sq_out):
    x = ef[...]
    wsd = srcw[...] - dstw[...]
    scale = scg[:, 0:1]
    pre = (jnp.dot(x, eloopw[...], preferred_element_type=jnp.float32)
           + scale * jnp.dot(x, wsd, preferred_element_type=jnp.float32)
           + (gd[...] - gs[...]) + eb[...])
    h = jnp.dot(pre, l1w[...], preferred_element_type=jnp.float32) + l1b[...]
    h_out[...] = h

    @pl.when(pl.program_id(0) == 0)
    def _():
        s_out[...] = jnp.zeros_like(s_out)
        sq_out[...] = jnp.zeros_like(sq_out)

    s_out[...] += jnp.sum(h, axis=0, keepdims=True)
    sq_out[...] += jnp.sum(h * h, axis=0, keepdims=True)


def _tc_e1(ef, gd, gs, scg, p):
    nsteps = E // BE
    row = lambda i: (i, 0)
    fixed = lambda i: (0, 0)
    return pl.pallas_call(
        _tc_e1_body,
        grid=(nsteps,),
        in_specs=[
            pl.BlockSpec((BE, D), row),
            pl.BlockSpec((BE, D), row),
            pl.BlockSpec((BE, D), row),
            pl.BlockSpec((BE, D), row),
            pl.BlockSpec((D, D), fixed),
            pl.BlockSpec((D, D), fixed),
            pl.BlockSpec((D, D), fixed),
            pl.BlockSpec((1, D), fixed),
            pl.BlockSpec((D, D), fixed),
            pl.BlockSpec((1, D), fixed),
        ],
        out_specs=[
            pl.BlockSpec((BE, D), row),
            pl.BlockSpec((1, D), fixed),
            pl.BlockSpec((1, D), fixed),
        ],
        out_shape=[
            jax.ShapeDtypeStruct((E, D), jnp.float32),
            jax.ShapeDtypeStruct((1, D), jnp.float32),
            jax.ShapeDtypeStruct((1, D), jnp.float32),
        ],
    )(ef, gd, gs, scg, p["eloop_w"], p["src_w"], p["dst_w"],
      p["ebias"].reshape(1, D), p["e_l1_w"], p["e_l1_b"].reshape(1, D))


def _tc_e2_body(h_ref, s_ref, sq_ref, g_ref, b_ref, l2w, l2b, out_ref):
    mu = s_ref[...] * (1.0 / E)
    var = sq_ref[...] * (1.0 / E) - mu * mu
    hn = (h_ref[...] - mu) * lax.rsqrt(var + 1e-5) * g_ref[...] + b_ref[...]
    out_ref[...] = (jnp.dot(jnp.maximum(hn, 0.0), l2w[...],
                            preferred_element_type=jnp.float32) + l2b[...])


def _tc_e2(h, s, sq, p):
    nsteps = E // BE
    row = lambda i: (i, 0)
    fixed = lambda i: (0, 0)
    return pl.pallas_call(
        _tc_e2_body,
        grid=(nsteps,),
        in_specs=[
            pl.BlockSpec((BE, D), row),
            pl.BlockSpec((1, D), fixed),
            pl.BlockSpec((1, D), fixed),
            pl.BlockSpec((1, D), fixed),
            pl.BlockSpec((1, D), fixed),
            pl.BlockSpec((D, D), fixed),
            pl.BlockSpec((1, D), fixed),
        ],
        out_specs=pl.BlockSpec((BE, D), row),
        out_shape=jax.ShapeDtypeStruct((E, D), jnp.float32),
    )(h, s, sq, p["e_bn_g"].reshape(1, D), p["e_bn_b"].reshape(1, D),
      p["e_l2_w"], p["e_l2_b"].reshape(1, D))


def _tc_node_body(nf, agg0, agg1, inw, nloopw, nb, l1w, l1b, g, b, l2w, l2b,
                  out_ref):
    x = nf[...]
    agg = jnp.concatenate([agg0[...], agg1[...]], axis=0)[:N]
    pre = (jnp.dot(x, nloopw[...], preferred_element_type=jnp.float32)
           - jnp.dot(agg, inw[...], preferred_element_type=jnp.float32)
           + nb[...])
    h = jnp.dot(pre, l1w[...], preferred_element_type=jnp.float32) + l1b[...]
    mu = jnp.mean(h, axis=0, keepdims=True)
    var = jnp.mean((h - mu) ** 2, axis=0, keepdims=True)
    hn = (h - mu) * lax.rsqrt(var + 1e-5) * g[...] + b[...]
    out_ref[...] = (jnp.dot(jnp.maximum(hn, 0.0), l2w[...],
                            preferred_element_type=jnp.float32) + l2b[...])


def _tc_node(nf, agg0, agg1, p):
    return pl.pallas_call(
        _tc_node_body,
        out_shape=jax.ShapeDtypeStruct((N, D), jnp.float32),
    )(nf, agg0, agg1, p["in_w"], p["nloop_w"], p["nbias"].reshape(1, D),
      p["n_l1_w"], p["n_l1_b"].reshape(1, D),
      p["n_bn_g"].reshape(1, D), p["n_bn_b"].reshape(1, D),
      p["n_l2_w"], p["n_l2_b"].reshape(1, D))


def _tc_lrp_body(gn, ge, wf, lb, pool_ref, out_ref):
    ps = gn[...] + ge[...]
    mm = jnp.dot(ps, wf[...], preferred_element_type=jnp.float32) + lb[...]
    pid = pool_ref[0, 0, :]
    oh = (pid[:, None] == lax.broadcasted_iota(jnp.int32, (BP, G), 1)
          ).astype(jnp.float32)
    contrib = lax.dot_general(oh, mm, (((0,), (0,)), ((), ())),
                              preferred_element_type=jnp.float32)

    @pl.when(pl.program_id(0) == 0)
    def _():
        out_ref[...] = jnp.zeros_like(out_ref)

    out_ref[...] += contrib


def _tc_lrp(gn2, ge2, wflat, lrp_bias, pool3):
    nsteps = P // BP
    row = lambda i: (i, 0)
    fixed = lambda i: (0, 0)
    return pl.pallas_call(
        _tc_lrp_body,
        grid=(nsteps,),
        in_specs=[
            pl.BlockSpec((BP, L2 * D), row),
            pl.BlockSpec((BP, L2 * D), row),
            pl.BlockSpec((L2 * D, D), fixed),
            pl.BlockSpec((1, D), fixed),
            pl.BlockSpec((1, 1, BP), lambda i: (i, 0, 0)),
        ],
        out_specs=pl.BlockSpec((G, D), fixed),
        out_shape=jax.ShapeDtypeStruct((G, D), jnp.float32),
    )(gn2, ge2, wflat, lrp_bias, pool3)


def kernel(node_feat, edge_feat, params, edge_index, node_perm_idx,
           edge_perm_idx, pool_idx):
    p = params
    src = edge_index[0]
    dst = edge_index[1]

    znd = jnp.zeros((NPAD4, D), jnp.float32)

    ones_blk = jnp.ones((KE, D), jnp.float32)
    agg0 = _sc_scatter_q0(edge_feat, dst, znd)
    agg1 = _sc_scatter_q1(edge_feat, dst, znd)
    deg0 = _sc_deg_q0(src, znd, ones_blk)
    deg1 = _sc_deg_q1(src, znd, ones_blk)
    nfd, nfs, stab = _tc_prep(node_feat, p["dst_w"], p["src_w"], deg0, deg1)
    gd, gs, scg = _sc_edge_gather(nfd, nfs, stab, src, dst)
    h, s, sq = _tc_e1(edge_feat, gd, gs, scg, p)
    edge_out = _tc_e2(h, s, sq, p)
    node_out = _tc_node(node_feat, agg0, agg1, p)
    gn, ge = _sc_perm_gather(node_out, edge_out, node_perm_idx,
                             edge_perm_idx)
    gn2 = gn.reshape(P, L2 * D)
    ge2 = ge.reshape(P, L2 * D)
    wflat = jnp.transpose(p["lrp_w"], (2, 0, 1)).reshape(L2 * D, D)
    pool3 = pool_idx.reshape(P // BP, 1, BP)
    pooled = _tc_lrp(gn2, ge2, wflat, p["lrp_bias"].reshape(1, D), pool3)
    return pooled, edge_out


# speedup vs baseline: 1.7327x; 1.0687x over previous
import functools

import jax
import jax.numpy as jnp
from jax import lax
from jax.experimental import pallas as pl
from jax.experimental.pallas import tpu as pltpu
from jax.experimental.pallas import tpu_sc as plsc

N = 10000
E = 320000
D = 128
L2 = 16
P = 50000
G = 256
PL = P * L2

NC = 2
NS = 16
NW = NC * NS
EPW = E // NW
KE = 400
NBE = EPW // KE
KEB = 200
NBEB = EPW // KEB
PPW = PL // NW
KP = 200
NBP = PPW // KP
NPAD = 10240
RPS = NPAD // NS

BE = 2560
BP = 400

_mesh = plsc.VectorSubcoreMesh(core_axis_name="c", subcore_axis_name="s")

EPW_A = E // NS
NBE_A = EPW_A // KE
NHALF = NPAD // 2
NQ = NPAD // 4
NPAD4 = 2688
RQ4 = NPAD4 // NS
WQ4 = NQ // NS


def _make_sc_scatter(q0):
    @functools.partial(
        pl.kernel,
        mesh=_mesh,
        out_type=jax.ShapeDtypeStruct((NHALF, D), jnp.float32),
        scratch_types=[
            pltpu.VMEM((KE,), jnp.int32),
            pltpu.VMEM((KE, D), jnp.float32),
            pltpu.VMEM_SHARED((NPAD4, D), jnp.float32),
            pltpu.SemaphoreType.DMA,
            pltpu.SemaphoreType.DMA,
        ],
        name=f"sc_scatter_q{q0}",
    )
    def _sc_scatter(ef_hbm, dst_hbm, znd_hbm, agg_hbm,
                    idxd, rowbuf, agg_sh, sem, sem2):
        c = lax.axis_index("c")
        s = lax.axis_index("s")
        lo = (2 * q0 + c) * NQ
        pltpu.sync_copy(znd_hbm.at[pl.ds(s * RQ4, RQ4)],
                        agg_sh.at[pl.ds(s * RQ4, RQ4)])
        plsc.subcore_barrier()

        def clamp(i, carry):
            sl = pl.ds(i * 16, 16)
            vd = idxd[sl] - lo
            okd = jnp.logical_and(vd >= 0, vd < NQ)
            idxd[sl] = jnp.where(okd, vd, NQ)
            return carry

        def blk(j, carry):
            base = s * EPW_A + j * KE
            a1 = pltpu.async_copy(dst_hbm.at[pl.ds(base, KE)], idxd, sem)
            a2 = pltpu.async_copy(ef_hbm.at[pl.ds(base, KE)], rowbuf, sem2)
            a1.wait()
            lax.fori_loop(0, KE // 16, clamp, 0)
            a2.wait()
            pltpu.sync_copy(rowbuf, agg_sh.at[idxd], add=True)
            return carry

        lax.fori_loop(0, NBE_A, blk, 0)
        plsc.subcore_barrier()
        pltpu.sync_copy(agg_sh.at[pl.ds(s * WQ4, WQ4)],
                        agg_hbm.at[pl.ds(c * NQ + s * WQ4, WQ4)])

    return _sc_scatter


_sc_scatter_q0 = _make_sc_scatter(0)
_sc_scatter_q1 = _make_sc_scatter(1)


def _make_sc_deg(q0):
    @functools.partial(
        pl.kernel,
        mesh=_mesh,
        out_type=jax.ShapeDtypeStruct((NHALF, D), jnp.float32),
        scratch_types=[
            pltpu.VMEM((KE,), jnp.int32),
            pltpu.VMEM((KE, D), jnp.float32),
            pltpu.VMEM_SHARED((NPAD4, D), jnp.float32),
        ],
        name=f"sc_deg_q{q0}",
    )
    def _sc_deg(src_hbm, znd_hbm, ones_hbm, deg_hbm, idxs, onesbuf, deg_sh):
        c = lax.axis_index("c")
        s = lax.axis_index("s")
        lo = (2 * q0 + c) * NQ
        pltpu.sync_copy(znd_hbm.at[pl.ds(s * RQ4, RQ4)],
                        deg_sh.at[pl.ds(s * RQ4, RQ4)])
        pltpu.sync_copy(ones_hbm, onesbuf)
        plsc.subcore_barrier()

        def clamp(i, carry):
            sl = pl.ds(i * 16, 16)
            vs = idxs[sl] - lo
            oks = jnp.logical_and(vs >= 0, vs < NQ)
            idxs[sl] = jnp.where(oks, vs, NQ)
            return carry

        def blk(j, carry):
            base = s * EPW_A + j * KE
            pltpu.sync_copy(src_hbm.at[pl.ds(base, KE)], idxs)
            lax.fori_loop(0, KE // 16, clamp, 0)
            pltpu.sync_copy(onesbuf, deg_sh.at[idxs], add=True)
            return carry

        lax.fori_loop(0, NBE_A, blk, 0)
        plsc.subcore_barrier()
        pltpu.sync_copy(deg_sh.at[pl.ds(s * WQ4, WQ4)],
                        deg_hbm.at[pl.ds(c * NQ + s * WQ4, WQ4)])

    return _sc_deg


_sc_deg_q0 = _make_sc_deg(0)
_sc_deg_q1 = _make_sc_deg(1)


@functools.partial(
    pl.kernel,
    mesh=_mesh,
    out_type=[
        jax.ShapeDtypeStruct((E, D), jnp.float32),
        jax.ShapeDtypeStruct((E, D), jnp.float32),
        jax.ShapeDtypeStruct((E, D), jnp.float32),
    ],
    scratch_types=[
        pltpu.VMEM((KEB,), jnp.int32),
        pltpu.VMEM((KEB,), jnp.int32),
        pltpu.VMEM((KEB, D), jnp.float32),
        pltpu.VMEM((KEB, D), jnp.float32),
        pltpu.VMEM((KEB, D), jnp.float32),
        pltpu.SemaphoreType.DMA,
        pltpu.SemaphoreType.DMA,
        pltpu.SemaphoreType.DMA,
    ],
)
def _sc_edge_gather(nfd_hbm, nfs_hbm, stab_hbm, src_hbm, dst_hbm,
                    gd_hbm, gs_hbm, scg_hbm,
                    idxd, idxs, bufd, bufs, bufsc, sem, sem2, sem3):
    c = lax.axis_index("c")
    s = lax.axis_index("s")
    wid = s * NC + c

    def blk(j, carry):
        base = wid * EPW + j * KEB
        a1 = pltpu.async_copy(dst_hbm.at[pl.ds(base, KEB)], idxd, sem)
        a2 = pltpu.async_copy(src_hbm.at[pl.ds(base, KEB)], idxs, sem2)
        a1.wait()
        a2.wait()
        g1 = pltpu.async_copy(nfd_hbm.at[idxd], bufd, sem)
        g2 = pltpu.async_copy(nfs_hbm.at[idxs], bufs, sem2)
        g3 = pltpu.async_copy(stab_hbm.at[idxd], bufsc, sem3)
        g1.wait()
        g2.wait()
        g3.wait()
        w1 = pltpu.async_copy(bufd, gd_hbm.at[pl.ds(base, KEB)], sem)
        w2 = pltpu.async_copy(bufs, gs_hbm.at[pl.ds(base, KEB)], sem2)
        w3 = pltpu.async_copy(bufsc, scg_hbm.at[pl.ds(base, KEB)], sem3)
        w1.wait()
        w2.wait()
        w3.wait()
        return carry

    lax.fori_loop(0, NBEB, blk, 0)


@functools.partial(
    pl.kernel,
    mesh=_mesh,
    out_type=[
        jax.ShapeDtypeStruct((PL, D), jnp.float32),
        jax.ShapeDtypeStruct((PL, D), jnp.float32),
    ],
    scratch_types=[
        pltpu.VMEM((KP,), jnp.int32),
        pltpu.VMEM((KP,), jnp.int32),
        pltpu.VMEM((KP, D), jnp.float32),
        pltpu.VMEM((KP, D), jnp.float32),
        pltpu.SemaphoreType.DMA,
        pltpu.SemaphoreType.DMA,
    ],
)
def _sc_perm_gather(nout_hbm, eout_hbm, npi_hbm, epi_hbm,
                    gn_hbm, ge_hbm,
                    idxn, idxe, bufn, bufe, sem, sem2):
    c = lax.axis_index("c")
    s = lax.axis_index("s")
    wid = s * NC + c

    def blk(j, carry):
        base = wid * PPW + j * KP
        a1 = pltpu.async_copy(npi_hbm.at[pl.ds(base, KP)], idxn, sem)
        a2 = pltpu.async_copy(epi_hbm.at[pl.ds(base, KP)], idxe, sem2)
        a1.wait()
        a2.wait()
        g1 = pltpu.async_copy(nout_hbm.at[idxn], bufn, sem)
        g2 = pltpu.async_copy(eout_hbm.at[idxe], bufe, sem2)
        g1.wait()
        g2.wait()
        w1 = pltpu.async_copy(bufn, gn_hbm.at[pl.ds(base, KP)], sem)
        w2 = pltpu.async_copy(bufe, ge_hbm.at[pl.ds(base, KP)], sem2)
        w1.wait()
        w2.wait()
        return carry

    lax.fori_loop(0, NBP, blk, 0)


def _tc_prep_body(nf, dstw, srcw, deg0, deg1, nfd, nfs, stab):
    x = nf[...]
    nfd[...] = jnp.dot(x, dstw[...], preferred_element_type=jnp.float32)
    nfs[...] = jnp.dot(x, srcw[...], preferred_element_type=jnp.float32)
    deg = jnp.concatenate([deg0[...], deg1[...]], axis=0)[:N, 0:1]
    sc = 2.0 * (1.0 + jnp.log(1.0 + deg) * 1.4426950408889634)
    stab[...] = jnp.broadcast_to(sc, (N, D))


def _tc_prep(nf, dstw, srcw, deg0, deg1):
    return pl.pallas_call(
        _tc_prep_body,
        out_shape=[
            jax.ShapeDtypeStruct((N, D), jnp.float32),
            jax.ShapeDtypeStruct((N, D), jnp.float32),
            jax.ShapeDtypeStruct((N, D), jnp.float32),
        ],
    )(nf, dstw, srcw, deg0, deg1)


def _tc_e1_body(ef, gd, gs, scg, eloopw, srcw, dstw, eb, l1w, l1b,
                h_out, s_out, sq_out):
    x = ef[...]
    wsd = srcw[...] - dstw[...]
    scale = scg[:, 0:1]
    pre = (jnp.dot(x, eloopw[...], preferred_element_type=jnp.float32)
           + scale * jnp.dot(x, wsd, preferred_element_type=jnp.float32)
           + (gd[...] - gs[...]) + eb[...])
    h = jnp.dot(pre, l1w[...], preferred_element_type=jnp.float32) + l1b[...]
    h_out[...] = h

    @pl.when(pl.program_id(0) == 0)
    def _():
        s_out[...] = jnp.zeros_like(s_out)
        sq_out[...] = jnp.zeros_like(sq_out)

    s_out[...] += jnp.sum(h, axis=0, keepdims=True)
    sq_out[...] += jnp.sum(h * h, axis=0, keepdims=True)


def _tc_e1(ef, gd, gs, scg, p):
    nsteps = E // BE
    row = lambda i: (i, 0)
    fixed = lambda i: (0, 0)
    return pl.pallas_call(
        _tc_e1_body,
        grid=(nsteps,),
        in_specs=[
            pl.BlockSpec((BE, D), row),
            pl.BlockSpec((BE, D), row),
            pl.BlockSpec((BE, D), row),
            pl.BlockSpec((BE, D), row),
            pl.BlockSpec((D, D), fixed),
            pl.BlockSpec((D, D), fixed),
            pl.BlockSpec((D, D), fixed),
            pl.BlockSpec((1, D), fixed),
            pl.BlockSpec((D, D), fixed),
            pl.BlockSpec((1, D), fixed),
        ],
        out_specs=[
            pl.BlockSpec((BE, D), row),
            pl.BlockSpec((1, D), fixed),
            pl.BlockSpec((1, D), fixed),
        ],
        out_shape=[
            jax.ShapeDtypeStruct((E, D), jnp.float32),
            jax.ShapeDtypeStruct((1, D), jnp.float32),
            jax.ShapeDtypeStruct((1, D), jnp.float32),
        ],
    )(ef, gd, gs, scg, p["eloop_w"], p["src_w"], p["dst_w"],
      p["ebias"].reshape(1, D), p["e_l1_w"], p["e_l1_b"].reshape(1, D))


def _tc_e2_body(h_ref, s_ref, sq_ref, g_ref, b_ref, l2w, l2b, out_ref):
    mu = s_ref[...] * (1.0 / E)
    var = sq_ref[...] * (1.0 / E) - mu * mu
    hn = (h_ref[...] - mu) * lax.rsqrt(var + 1e-5) * g_ref[...] + b_ref[...]
    out_ref[...] = (jnp.dot(jnp.maximum(hn, 0.0), l2w[...],
                            preferred_element_type=jnp.float32) + l2b[...])


def _tc_e2(h, s, sq, p):
    nsteps = E // BE
    row = lambda i: (i, 0)
    fixed = lambda i: (0, 0)
    return pl.pallas_call(
        _tc_e2_body,
        grid=(nsteps,),
        in_specs=[
            pl.BlockSpec((BE, D), row),
            pl.BlockSpec((1, D), fixed),
            pl.BlockSpec((1, D), fixed),
            pl.BlockSpec((1, D), fixed),
            pl.BlockSpec((1, D), fixed),
            pl.BlockSpec((D, D), fixed),
            pl.BlockSpec((1, D), fixed),
        ],
        out_specs=pl.BlockSpec((BE, D), row),
        out_shape=jax.ShapeDtypeStruct((E, D), jnp.float32),
    )(h, s, sq, p["e_bn_g"].reshape(1, D), p["e_bn_b"].reshape(1, D),
      p["e_l2_w"], p["e_l2_b"].reshape(1, D))


def _tc_node_body(nf, agg0, agg1, inw, nloopw, nb, l1w, l1b, g, b, l2w, l2b,
                  out_ref):
    x = nf[...]
    agg = jnp.concatenate([agg0[...], agg1[...]], axis=0)[:N]
    pre = (jnp.dot(x, nloopw[...], preferred_element_type=jnp.float32)
           - jnp.dot(agg, inw[...], preferred_element_type=jnp.float32)
           + nb[...])
    h = jnp.dot(pre, l1w[...], preferred_element_type=jnp.float32) + l1b[...]
    mu = jnp.mean(h, axis=0, keepdims=True)
    var = jnp.mean((h - mu) ** 2, axis=0, keepdims=True)
    hn = (h - mu) * lax.rsqrt(var + 1e-5) * g[...] + b[...]
    out_ref[...] = (jnp.dot(jnp.maximum(hn, 0.0), l2w[...],
                            preferred_element_type=jnp.float32) + l2b[...])


def _tc_node(nf, agg0, agg1, p):
    return pl.pallas_call(
        _tc_node_body,
        out_shape=jax.ShapeDtypeStruct((N, D), jnp.float32),
    )(nf, agg0, agg1, p["in_w"], p["nloop_w"], p["nbias"].reshape(1, D),
      p["n_l1_w"], p["n_l1_b"].reshape(1, D),
      p["n_bn_g"].reshape(1, D), p["n_bn_b"].reshape(1, D),
      p["n_l2_w"], p["n_l2_b"].reshape(1, D))


def _tc_lrp_body(gn, ge, wf, lb, pool_ref, out_ref):
    ps = gn[...] + ge[...]
    mm = jnp.dot(ps, wf[...], preferred_element_type=jnp.float32) + lb[...]
    pid = pool_ref[0, 0, :]
    oh = (pid[:, None] == lax.broadcasted_iota(jnp.int32, (BP, G), 1)
          ).astype(jnp.float32)
    contrib = lax.dot_general(oh, mm, (((0,), (0,)), ((), ())),
                              preferred_element_type=jnp.float32)

    @pl.when(pl.program_id(0) == 0)
    def _():
        out_ref[...] = jnp.zeros_like(out_ref)

    out_ref[...] += contrib


def _tc_lrp(gn2, ge2, wflat, lrp_bias, pool3):
    nsteps = P // BP
    row = lambda i: (i, 0)
    fixed = lambda i: (0, 0)
    return pl.pallas_call(
        _tc_lrp_body,
        grid=(nsteps,),
        in_specs=[
            pl.BlockSpec((BP, L2 * D), row),
            pl.BlockSpec((BP, L2 * D), row),
            pl.BlockSpec((L2 * D, D), fixed),
            pl.BlockSpec((1, D), fixed),
            pl.BlockSpec((1, 1, BP), lambda i: (i, 0, 0)),
        ],
        out_specs=pl.BlockSpec((G, D), fixed),
        out_shape=jax.ShapeDtypeStruct((G, D), jnp.float32),
    )(gn2, ge2, wflat, lrp_bias, pool3)


def kernel(node_feat, edge_feat, params, edge_index, node_perm_idx,
           edge_perm_idx, pool_idx):
    p = params
    src = edge_index[0]
    dst = edge_index[1]

    znd = jnp.zeros((NPAD4, D), jnp.float32)

    ones_blk = jnp.ones((KE, D), jnp.float32)
    agg0 = _sc_scatter_q0(edge_feat, dst, znd)
    agg1 = _sc_scatter_q1(edge_feat, dst, znd)
    deg0 = _sc_deg_q0(src, znd, ones_blk)
    deg1 = _sc_deg_q1(src, znd, ones_blk)
    nfd, nfs, stab = _tc_prep(node_feat, p["dst_w"], p["src_w"], deg0, deg1)
    gd, gs, scg = _sc_edge_gather(nfd, nfs, stab, src, dst)
    h, s, sq = _tc_e1(edge_feat, gd, gs, scg, p)
    edge_out = _tc_e2(h, s, sq, p)
    node_out = _tc_node(node_feat, agg0, agg1, p)
    gn, ge = _sc_perm_gather(node_out, edge_out, node_perm_idx,
                             edge_perm_idx)
    gn2 = gn.reshape(P, L2 * D)
    ge2 = ge.reshape(P, L2 * D)
    wflat = jnp.transpose(p["lrp_w"], (2, 0, 1)).reshape(L2 * D, D)
    pool3 = pool_idx.reshape(P // BP, 1, BP)
    pooled = _tc_lrp(gn2, ge2, wflat, p["lrp_bias"].reshape(1, D), pool3)
    return pooled, edge_out


# speedup vs baseline: 1.9239x; 1.1104x over previous
import functools

import jax
import jax.numpy as jnp
from jax import lax
from jax.experimental import pallas as pl
from jax.experimental.pallas import tpu as pltpu
from jax.experimental.pallas import tpu_sc as plsc

N = 10000
E = 320000
D = 128
L2 = 16
P = 50000
G = 256
PL = P * L2

NC = 2
NS = 16
NW = NC * NS
EPW = E // NW
KE = 400
NBE = EPW // KE
KEB = 200
NBEB = EPW // KEB
PPW = PL // NW
KP = 200
NBP = PPW // KP
NPAD = 10240
RPS = NPAD // NS

BE = 2560
BP = 400

_mesh = plsc.VectorSubcoreMesh(core_axis_name="c", subcore_axis_name="s")

EPW_A = E // NS
NBE_A = EPW_A // KE
NHALF = NPAD // 2
NQ = NPAD // 4
NPAD4 = 2688
RQ4 = NPAD4 // NS
WQ4 = NQ // NS


def _make_sc_scatter(q0):
    @functools.partial(
        pl.kernel,
        mesh=_mesh,
        out_type=jax.ShapeDtypeStruct((NHALF, D), jnp.float32),
        scratch_types=[
            pltpu.VMEM((KE,), jnp.int32),
            pltpu.VMEM((KE, D), jnp.float32),
            pltpu.VMEM_SHARED((NPAD4, D), jnp.float32),
            pltpu.SemaphoreType.DMA,
            pltpu.SemaphoreType.DMA,
        ],
        name=f"sc_scatter_q{q0}",
    )
    def _sc_scatter(ef_hbm, dst_hbm, znd_hbm, agg_hbm,
                    idxd, rowbuf, agg_sh, sem, sem2):
        c = lax.axis_index("c")
        s = lax.axis_index("s")
        lo = (2 * q0 + c) * NQ
        pltpu.sync_copy(znd_hbm.at[pl.ds(s * RQ4, RQ4)],
                        agg_sh.at[pl.ds(s * RQ4, RQ4)])
        plsc.subcore_barrier()

        def clamp(i, carry):
            sl = pl.ds(i * 16, 16)
            vd = idxd[sl] - lo
            okd = jnp.logical_and(vd >= 0, vd < NQ)
            idxd[sl] = jnp.where(okd, vd, NQ)
            return carry

        def blk(j, carry):
            base = s * EPW_A + j * KE
            a1 = pltpu.async_copy(dst_hbm.at[pl.ds(base, KE)], idxd, sem)
            a2 = pltpu.async_copy(ef_hbm.at[pl.ds(base, KE)], rowbuf, sem2)
            a1.wait()
            lax.fori_loop(0, KE // 16, clamp, 0)
            a2.wait()
            pltpu.sync_copy(rowbuf, agg_sh.at[idxd], add=True)
            return carry

        lax.fori_loop(0, NBE_A, blk, 0)
        plsc.subcore_barrier()
        pltpu.sync_copy(agg_sh.at[pl.ds(s * WQ4, WQ4)],
                        agg_hbm.at[pl.ds(c * NQ + s * WQ4, WQ4)])

    return _sc_scatter


_sc_scatter_q0 = _make_sc_scatter(0)
_sc_scatter_q1 = _make_sc_scatter(1)


def _make_sc_deg(q0):
    @functools.partial(
        pl.kernel,
        mesh=_mesh,
        out_type=jax.ShapeDtypeStruct((NHALF, D), jnp.float32),
        scratch_types=[
            pltpu.VMEM((KE,), jnp.int32),
            pltpu.VMEM((KE, D), jnp.float32),
            pltpu.VMEM_SHARED((NPAD4, D), jnp.float32),
        ],
        name=f"sc_deg_q{q0}",
    )
    def _sc_deg(src_hbm, znd_hbm, ones_hbm, deg_hbm, idxs, onesbuf, deg_sh):
        c = lax.axis_index("c")
        s = lax.axis_index("s")
        lo = (2 * q0 + c) * NQ
        pltpu.sync_copy(znd_hbm.at[pl.ds(s * RQ4, RQ4)],
                        deg_sh.at[pl.ds(s * RQ4, RQ4)])
        pltpu.sync_copy(ones_hbm, onesbuf)
        plsc.subcore_barrier()

        def clamp(i, carry):
            sl = pl.ds(i * 16, 16)
            vs = idxs[sl] - lo
            oks = jnp.logical_and(vs >= 0, vs < NQ)
            idxs[sl] = jnp.where(oks, vs, NQ)
            return carry

        def blk(j, carry):
            base = s * EPW_A + j * KE
            pltpu.sync_copy(src_hbm.at[pl.ds(base, KE)], idxs)
            lax.fori_loop(0, KE // 16, clamp, 0)
            pltpu.sync_copy(onesbuf, deg_sh.at[idxs], add=True)
            return carry

        lax.fori_loop(0, NBE_A, blk, 0)
        plsc.subcore_barrier()
        pltpu.sync_copy(deg_sh.at[pl.ds(s * WQ4, WQ4)],
                        deg_hbm.at[pl.ds(c * NQ + s * WQ4, WQ4)])

    return _sc_deg


_sc_deg_q0 = _make_sc_deg(0)
_sc_deg_q1 = _make_sc_deg(1)


@functools.partial(
    pl.kernel,
    mesh=_mesh,
    out_type=[
        jax.ShapeDtypeStruct((E, D), jnp.float32),
        jax.ShapeDtypeStruct((E, D), jnp.float32),
        jax.ShapeDtypeStruct((E, D), jnp.float32),
    ],
    scratch_types=[
        pltpu.VMEM((KEB,), jnp.int32),
        pltpu.VMEM((KEB,), jnp.int32),
        pltpu.VMEM((KEB, D), jnp.float32),
        pltpu.VMEM((KEB, D), jnp.float32),
        pltpu.VMEM((KEB, D), jnp.float32),
        pltpu.SemaphoreType.DMA,
        pltpu.SemaphoreType.DMA,
        pltpu.SemaphoreType.DMA,
    ],
)
def _sc_edge_gather(nfd_hbm, nfs_hbm, stab_hbm, src_hbm, dst_hbm,
                    gd_hbm, gs_hbm, scg_hbm,
                    idxd, idxs, bufd, bufs, bufsc, sem, sem2, sem3):
    c = lax.axis_index("c")
    s = lax.axis_index("s")
    wid = s * NC + c

    def blk(j, carry):
        base = wid * EPW + j * KEB
        a1 = pltpu.async_copy(dst_hbm.at[pl.ds(base, KEB)], idxd, sem)
        a2 = pltpu.async_copy(src_hbm.at[pl.ds(base, KEB)], idxs, sem2)
        a1.wait()
        a2.wait()
        g1 = pltpu.async_copy(nfd_hbm.at[idxd], bufd, sem)
        g2 = pltpu.async_copy(nfs_hbm.at[idxs], bufs, sem2)
        g3 = pltpu.async_copy(stab_hbm.at[idxd], bufsc, sem3)
        g1.wait()
        g2.wait()
        g3.wait()
        w1 = pltpu.async_copy(bufd, gd_hbm.at[pl.ds(base, KEB)], sem)
        w2 = pltpu.async_copy(bufs, gs_hbm.at[pl.ds(base, KEB)], sem2)
        w3 = pltpu.async_copy(bufsc, scg_hbm.at[pl.ds(base, KEB)], sem3)
        w1.wait()
        w2.wait()
        w3.wait()
        return carry

    lax.fori_loop(0, NBEB, blk, 0)


@functools.partial(
    pl.kernel,
    mesh=_mesh,
    out_type=jax.ShapeDtypeStruct((PL, D), jnp.float32),
    scratch_types=[
        pltpu.VMEM((KP,), jnp.int32),
        pltpu.VMEM((KP,), jnp.int32),
        pltpu.VMEM((KP, D), jnp.float32),
        pltpu.VMEM((KP, D), jnp.float32),
        pltpu.SemaphoreType.DMA,
        pltpu.SemaphoreType.DMA,
    ],
)
def _sc_perm_gather(nout_hbm, eout_hbm, npi_hbm, epi_hbm,
                    ps_hbm,
                    idxn, idxe, bufn, bufe, sem, sem2):
    c = lax.axis_index("c")
    s = lax.axis_index("s")
    wid = s * NC + c

    def addrow(r, carry):
        for cc in range(8):
            sl = pl.ds(cc * 16, 16)
            bufn[r, sl] = bufn[r, sl] + bufe[r, sl]
        return carry

    def blk(j, carry):
        base = wid * PPW + j * KP
        a1 = pltpu.async_copy(npi_hbm.at[pl.ds(base, KP)], idxn, sem)
        a2 = pltpu.async_copy(epi_hbm.at[pl.ds(base, KP)], idxe, sem2)
        a1.wait()
        a2.wait()
        g1 = pltpu.async_copy(nout_hbm.at[idxn], bufn, sem)
        g2 = pltpu.async_copy(eout_hbm.at[idxe], bufe, sem2)
        g1.wait()
        g2.wait()
        lax.fori_loop(0, KP, addrow, 0)
        pltpu.sync_copy(bufn, ps_hbm.at[pl.ds(base, KP)])
        return carry

    lax.fori_loop(0, NBP, blk, 0)


def _tc_prep_body(nf, dstw, srcw, deg0, deg1, nfd, nfs, stab):
    x = nf[...]
    nfd[...] = jnp.dot(x, dstw[...], preferred_element_type=jnp.float32)
    nfs[...] = jnp.dot(x, srcw[...], preferred_element_type=jnp.float32)
    deg = jnp.concatenate([deg0[...], deg1[...]], axis=0)[:N, 0:1]
    sc = 2.0 * (1.0 + jnp.log(1.0 + deg) * 1.4426950408889634)
    stab[...] = jnp.broadcast_to(sc, (N, D))


def _tc_prep(nf, dstw, srcw, deg0, deg1):
    return pl.pallas_call(
        _tc_prep_body,
        out_shape=[
            jax.ShapeDtypeStruct((N, D), jnp.float32),
            jax.ShapeDtypeStruct((N, D), jnp.float32),
            jax.ShapeDtypeStruct((N, D), jnp.float32),
        ],
    )(nf, dstw, srcw, deg0, deg1)


def _tc_e1_body(ef, gd, gs, scg, eloopw, srcw, dstw, eb, l1w, l1b,
                h_out, s_out, sq_out):
    x = ef[...]
    wsd = srcw[...] - dstw[...]
    scale = scg[:, 0:1]
    pre = (jnp.dot(x, eloopw[...], preferred_element_type=jnp.float32)
           + scale * jnp.dot(x, wsd, preferred_element_type=jnp.float32)
           + (gd[...] - gs[...]) + eb[...])
    h = jnp.dot(pre, l1w[...], preferred_element_type=jnp.float32) + l1b[...]
    h_out[...] = h

    @pl.when(pl.program_id(0) == 0)
    def _():
        s_out[...] = jnp.zeros_like(s_out)
        sq_out[...] = jnp.zeros_like(sq_out)

    s_out[...] += jnp.sum(h, axis=0, keepdims=True)
    sq_out[...] += jnp.sum(h * h, axis=0, keepdims=True)


def _tc_e1(ef, gd, gs, scg, p):
    nsteps = E // BE
    row = lambda i: (i, 0)
    fixed = lambda i: (0, 0)
    return pl.pallas_call(
        _tc_e1_body,
        grid=(nsteps,),
        in_specs=[
            pl.BlockSpec((BE, D), row),
            pl.BlockSpec((BE, D), row),
            pl.BlockSpec((BE, D), row),
            pl.BlockSpec((BE, D), row),
            pl.BlockSpec((D, D), fixed),
            pl.BlockSpec((D, D), fixed),
            pl.BlockSpec((D, D), fixed),
            pl.BlockSpec((1, D), fixed),
            pl.BlockSpec((D, D), fixed),
            pl.BlockSpec((1, D), fixed),
        ],
        out_specs=[
            pl.BlockSpec((BE, D), row),
            pl.BlockSpec((1, D), fixed),
            pl.BlockSpec((1, D), fixed),
        ],
        out_shape=[
            jax.ShapeDtypeStruct((E, D), jnp.float32),
            jax.ShapeDtypeStruct((1, D), jnp.float32),
            jax.ShapeDtypeStruct((1, D), jnp.float32),
        ],
    )(ef, gd, gs, scg, p["eloop_w"], p["src_w"], p["dst_w"],
      p["ebias"].reshape(1, D), p["e_l1_w"], p["e_l1_b"].reshape(1, D))


def _tc_e2_body(h_ref, s_ref, sq_ref, g_ref, b_ref, l2w, l2b, out_ref):
    mu = s_ref[...] * (1.0 / E)
    var = sq_ref[...] * (1.0 / E) - mu * mu
    hn = (h_ref[...] - mu) * lax.rsqrt(var + 1e-5) * g_ref[...] + b_ref[...]
    out_ref[...] = (jnp.dot(jnp.maximum(hn, 0.0), l2w[...],
                            preferred_element_type=jnp.float32) + l2b[...])


def _tc_e2(h, s, sq, p):
    nsteps = E // BE
    row = lambda i: (i, 0)
    fixed = lambda i: (0, 0)
    return pl.pallas_call(
        _tc_e2_body,
        grid=(nsteps,),
        in_specs=[
            pl.BlockSpec((BE, D), row),
            pl.BlockSpec((1, D), fixed),
            pl.BlockSpec((1, D), fixed),
            pl.BlockSpec((1, D), fixed),
            pl.BlockSpec((1, D), fixed),
            pl.BlockSpec((D, D), fixed),
            pl.BlockSpec((1, D), fixed),
        ],
        out_specs=pl.BlockSpec((BE, D), row),
        out_shape=jax.ShapeDtypeStruct((E, D), jnp.float32),
    )(h, s, sq, p["e_bn_g"].reshape(1, D), p["e_bn_b"].reshape(1, D),
      p["e_l2_w"], p["e_l2_b"].reshape(1, D))


def _tc_node_body(nf, agg0, agg1, inw, nloopw, nb, l1w, l1b, g, b, l2w, l2b,
                  out_ref):
    x = nf[...]
    agg = jnp.concatenate([agg0[...], agg1[...]], axis=0)[:N]
    pre = (jnp.dot(x, nloopw[...], preferred_element_type=jnp.float32)
           - jnp.dot(agg, inw[...], preferred_element_type=jnp.float32)
           + nb[...])
    h = jnp.dot(pre, l1w[...], preferred_element_type=jnp.float32) + l1b[...]
    mu = jnp.mean(h, axis=0, keepdims=True)
    var = jnp.mean((h - mu) ** 2, axis=0, keepdims=True)
    hn = (h - mu) * lax.rsqrt(var + 1e-5) * g[...] + b[...]
    out_ref[...] = (jnp.dot(jnp.maximum(hn, 0.0), l2w[...],
                            preferred_element_type=jnp.float32) + l2b[...])


def _tc_node(nf, agg0, agg1, p):
    return pl.pallas_call(
        _tc_node_body,
        out_shape=jax.ShapeDtypeStruct((N, D), jnp.float32),
    )(nf, agg0, agg1, p["in_w"], p["nloop_w"], p["nbias"].reshape(1, D),
      p["n_l1_w"], p["n_l1_b"].reshape(1, D),
      p["n_bn_g"].reshape(1, D), p["n_bn_b"].reshape(1, D),
      p["n_l2_w"], p["n_l2_b"].reshape(1, D))


def _tc_lrp_body(gn, wf, lb, pool_ref, out_ref):
    ps = gn[...]
    mm = jnp.dot(ps, wf[...], preferred_element_type=jnp.float32) + lb[...]
    pid = pool_ref[0, 0, :]
    oh = (pid[:, None] == lax.broadcasted_iota(jnp.int32, (BP, G), 1)
          ).astype(jnp.float32)
    contrib = lax.dot_general(oh, mm, (((0,), (0,)), ((), ())),
                              preferred_element_type=jnp.float32)

    @pl.when(pl.program_id(0) == 0)
    def _():
        out_ref[...] = jnp.zeros_like(out_ref)

    out_ref[...] += contrib


def _tc_lrp(gn2, wflat, lrp_bias, pool3):
    nsteps = P // BP
    row = lambda i: (i, 0)
    fixed = lambda i: (0, 0)
    return pl.pallas_call(
        _tc_lrp_body,
        grid=(nsteps,),
        in_specs=[
            pl.BlockSpec((BP, L2 * D), row),
            pl.BlockSpec((L2 * D, D), fixed),
            pl.BlockSpec((1, D), fixed),
            pl.BlockSpec((1, 1, BP), lambda i: (i, 0, 0)),
        ],
        out_specs=pl.BlockSpec((G, D), fixed),
        out_shape=jax.ShapeDtypeStruct((G, D), jnp.float32),
    )(gn2, wflat, lrp_bias, pool3)


def kernel(node_feat, edge_feat, params, edge_index, node_perm_idx,
           edge_perm_idx, pool_idx):
    p = params
    src = edge_index[0]
    dst = edge_index[1]

    znd = jnp.zeros((NPAD4, D), jnp.float32)

    ones_blk = jnp.ones((KE, D), jnp.float32)
    agg0 = _sc_scatter_q0(edge_feat, dst, znd)
    agg1 = _sc_scatter_q1(edge_feat, dst, znd)
    deg0 = _sc_deg_q0(src, znd, ones_blk)
    deg1 = _sc_deg_q1(src, znd, ones_blk)
    nfd, nfs, stab = _tc_prep(node_feat, p["dst_w"], p["src_w"], deg0, deg1)
    gd, gs, scg = _sc_edge_gather(nfd, nfs, stab, src, dst)
    h, s, sq = _tc_e1(edge_feat, gd, gs, scg, p)
    edge_out = _tc_e2(h, s, sq, p)
    node_out = _tc_node(node_feat, agg0, agg1, p)
    psum = _sc_perm_gather(node_out, edge_out, node_perm_idx,
                           edge_perm_idx)
    ps2 = psum.reshape(P, L2 * D)
    wflat = jnp.transpose(p["lrp_w"], (2, 0, 1)).reshape(L2 * D, D)
    pool3 = pool_idx.reshape(P // BP, 1, BP)
    pooled = _tc_lrp(ps2, wflat, p["lrp_bias"].reshape(1, D), pool3)
    return pooled, edge_out


# speedup vs baseline: 1.9251x; 1.0006x over previous
import functools

import jax
import jax.numpy as jnp
from jax import lax
from jax.experimental import pallas as pl
from jax.experimental.pallas import tpu as pltpu
from jax.experimental.pallas import tpu_sc as plsc

N = 10000
E = 320000
D = 128
L2 = 16
P = 50000
G = 256
PL = P * L2

NC = 2
NS = 16
NW = NC * NS
EPW = E // NW
KE = 400
NBE = EPW // KE
KEB = 200
NBEB = EPW // KEB
PPW = PL // NW
KP = 200
NBP = PPW // KP
NPAD = 10240
RPS = NPAD // NS

BE = 2560
BP = 400

_mesh = plsc.VectorSubcoreMesh(core_axis_name="c", subcore_axis_name="s")

EPW_A = E // NS
NBE_A = EPW_A // KE
NHALF = NPAD // 2
NQ = NPAD // 4
NPAD4 = 2688
RQ4 = NPAD4 // NS
WQ4 = NQ // NS


def _make_sc_scatter(q0):
    @functools.partial(
        pl.kernel,
        mesh=_mesh,
        out_type=jax.ShapeDtypeStruct((NHALF, D), jnp.float32),
        scratch_types=[
            pltpu.VMEM((KE,), jnp.int32),
            pltpu.VMEM((KE, D), jnp.float32),
            pltpu.VMEM_SHARED((NPAD4, D), jnp.float32),
            pltpu.SemaphoreType.DMA,
            pltpu.SemaphoreType.DMA,
        ],
        name=f"sc_scatter_q{q0}",
    )
    def _sc_scatter(ef_hbm, dst_hbm, znd_hbm, agg_hbm,
                    idxd, rowbuf, agg_sh, sem, sem2):
        c = lax.axis_index("c")
        s = lax.axis_index("s")
        lo = (2 * q0 + c) * NQ
        pltpu.sync_copy(znd_hbm.at[pl.ds(s * RQ4, RQ4)],
                        agg_sh.at[pl.ds(s * RQ4, RQ4)])
        plsc.subcore_barrier()

        def clamp(i, carry):
            sl = pl.ds(i * 16, 16)
            vd = idxd[sl] - lo
            okd = jnp.logical_and(vd >= 0, vd < NQ)
            idxd[sl] = jnp.where(okd, vd, NQ)
            return carry

        def blk(j, carry):
            base = s * EPW_A + j * KE
            a1 = pltpu.async_copy(dst_hbm.at[pl.ds(base, KE)], idxd, sem)
            a2 = pltpu.async_copy(ef_hbm.at[pl.ds(base, KE)], rowbuf, sem2)
            a1.wait()
            lax.fori_loop(0, KE // 16, clamp, 0)
            a2.wait()
            pltpu.sync_copy(rowbuf, agg_sh.at[idxd], add=True)
            return carry

        lax.fori_loop(0, NBE_A, blk, 0)
        plsc.subcore_barrier()
        pltpu.sync_copy(agg_sh.at[pl.ds(s * WQ4, WQ4)],
                        agg_hbm.at[pl.ds(c * NQ + s * WQ4, WQ4)])

    return _sc_scatter


_sc_scatter_q0 = _make_sc_scatter(0)
_sc_scatter_q1 = _make_sc_scatter(1)


def _make_sc_deg(q0):
    @functools.partial(
        pl.kernel,
        mesh=_mesh,
        out_type=jax.ShapeDtypeStruct((NHALF, D), jnp.float32),
        scratch_types=[
            pltpu.VMEM((KE,), jnp.int32),
            pltpu.VMEM((KE, D), jnp.float32),
            pltpu.VMEM_SHARED((NPAD4, D), jnp.float32),
        ],
        name=f"sc_deg_q{q0}",
    )
    def _sc_deg(src_hbm, znd_hbm, ones_hbm, deg_hbm, idxs, onesbuf, deg_sh):
        c = lax.axis_index("c")
        s = lax.axis_index("s")
        lo = (2 * q0 + c) * NQ
        pltpu.sync_copy(znd_hbm.at[pl.ds(s * RQ4, RQ4)],
                        deg_sh.at[pl.ds(s * RQ4, RQ4)])
        pltpu.sync_copy(ones_hbm, onesbuf)
        plsc.subcore_barrier()

        def clamp(i, carry):
            sl = pl.ds(i * 16, 16)
            vs = idxs[sl] - lo
            oks = jnp.logical_and(vs >= 0, vs < NQ)
            idxs[sl] = jnp.where(oks, vs, NQ)
            return carry

        def blk(j, carry):
            base = s * EPW_A + j * KE
            pltpu.sync_copy(src_hbm.at[pl.ds(base, KE)], idxs)
            lax.fori_loop(0, KE // 16, clamp, 0)
            pltpu.sync_copy(onesbuf, deg_sh.at[idxs], add=True)
            return carry

        lax.fori_loop(0, NBE_A, blk, 0)
        plsc.subcore_barrier()
        pltpu.sync_copy(deg_sh.at[pl.ds(s * WQ4, WQ4)],
                        deg_hbm.at[pl.ds(c * NQ + s * WQ4, WQ4)])

    return _sc_deg


_sc_deg_q0 = _make_sc_deg(0)
_sc_deg_q1 = _make_sc_deg(1)


@functools.partial(
    pl.kernel,
    mesh=_mesh,
    out_type=[
        jax.ShapeDtypeStruct((E, D), jnp.float32),
        jax.ShapeDtypeStruct((E, D), jnp.float32),
    ],
    scratch_types=[
        pltpu.VMEM((KEB,), jnp.int32),
        pltpu.VMEM((KEB,), jnp.int32),
        pltpu.VMEM((KEB, D), jnp.float32),
        pltpu.VMEM((KEB, D), jnp.float32),
        pltpu.VMEM((KEB, D), jnp.float32),
        pltpu.SemaphoreType.DMA,
        pltpu.SemaphoreType.DMA,
        pltpu.SemaphoreType.DMA,
    ],
)
def _sc_edge_gather(nfd_hbm, nfs_hbm, stab_hbm, src_hbm, dst_hbm,
                    emsg_hbm, scg_hbm,
                    idxd, idxs, bufd, bufs, bufsc, sem, sem2, sem3):
    c = lax.axis_index("c")
    s = lax.axis_index("s")
    wid = s * NC + c

    def subrow(r, carry):
        for cc in range(8):
            sl = pl.ds(cc * 16, 16)
            bufd[r, sl] = bufd[r, sl] - bufs[r, sl]
        return carry

    def blk(j, carry):
        base = wid * EPW + j * KEB
        a1 = pltpu.async_copy(dst_hbm.at[pl.ds(base, KEB)], idxd, sem)
        a2 = pltpu.async_copy(src_hbm.at[pl.ds(base, KEB)], idxs, sem2)
        a1.wait()
        a2.wait()
        g1 = pltpu.async_copy(nfd_hbm.at[idxd], bufd, sem)
        g2 = pltpu.async_copy(nfs_hbm.at[idxs], bufs, sem2)
        g3 = pltpu.async_copy(stab_hbm.at[idxd], bufsc, sem3)
        g1.wait()
        g2.wait()
        g3.wait()
        lax.fori_loop(0, KEB, subrow, 0)
        w1 = pltpu.async_copy(bufd, emsg_hbm.at[pl.ds(base, KEB)], sem)
        w3 = pltpu.async_copy(bufsc, scg_hbm.at[pl.ds(base, KEB)], sem3)
        w1.wait()
        w3.wait()
        return carry

    lax.fori_loop(0, NBEB, blk, 0)


@functools.partial(
    pl.kernel,
    mesh=_mesh,
    out_type=jax.ShapeDtypeStruct((PL, D), jnp.float32),
    scratch_types=[
        pltpu.VMEM((KP,), jnp.int32),
        pltpu.VMEM((KP,), jnp.int32),
        pltpu.VMEM((KP, D), jnp.float32),
        pltpu.VMEM((KP, D), jnp.float32),
        pltpu.SemaphoreType.DMA,
        pltpu.SemaphoreType.DMA,
    ],
)
def _sc_perm_gather(nout_hbm, eout_hbm, npi_hbm, epi_hbm,
                    ps_hbm,
                    idxn, idxe, bufn, bufe, sem, sem2):
    c = lax.axis_index("c")
    s = lax.axis_index("s")
    wid = s * NC + c

    def addrow(r, carry):
        for cc in range(8):
            sl = pl.ds(cc * 16, 16)
            bufn[r, sl] = bufn[r, sl] + bufe[r, sl]
        return carry

    def blk(j, carry):
        base = wid * PPW + j * KP
        a1 = pltpu.async_copy(npi_hbm.at[pl.ds(base, KP)], idxn, sem)
        a2 = pltpu.async_copy(epi_hbm.at[pl.ds(base, KP)], idxe, sem2)
        a1.wait()
        a2.wait()
        g1 = pltpu.async_copy(nout_hbm.at[idxn], bufn, sem)
        g2 = pltpu.async_copy(eout_hbm.at[idxe], bufe, sem2)
        g1.wait()
        g2.wait()
        lax.fori_loop(0, KP, addrow, 0)
        pltpu.sync_copy(bufn, ps_hbm.at[pl.ds(base, KP)])
        return carry

    lax.fori_loop(0, NBP, blk, 0)


def _tc_prep_body(nf, dstw, srcw, deg0, deg1, nfd, nfs, stab):
    x = nf[...]
    nfd[...] = jnp.dot(x, dstw[...], preferred_element_type=jnp.float32)
    nfs[...] = jnp.dot(x, srcw[...], preferred_element_type=jnp.float32)
    deg = jnp.concatenate([deg0[...], deg1[...]], axis=0)[:N, 0:1]
    sc = 2.0 * (1.0 + jnp.log(1.0 + deg) * 1.4426950408889634)
    stab[...] = jnp.broadcast_to(sc, (N, D))


def _tc_prep(nf, dstw, srcw, deg0, deg1):
    return pl.pallas_call(
        _tc_prep_body,
        out_shape=[
            jax.ShapeDtypeStruct((N, D), jnp.float32),
            jax.ShapeDtypeStruct((N, D), jnp.float32),
            jax.ShapeDtypeStruct((N, D), jnp.float32),
        ],
    )(nf, dstw, srcw, deg0, deg1)


def _tc_e1_body(ef, emsg, scg, eloopw, srcw, dstw, eb, l1w, l1b,
                h_out, s_out, sq_out):
    x = ef[...]
    wsd = srcw[...] - dstw[...]
    scale = scg[:, 0:1]
    pre = (jnp.dot(x, eloopw[...], preferred_element_type=jnp.float32)
           + scale * jnp.dot(x, wsd, preferred_element_type=jnp.float32)
           + emsg[...] + eb[...])
    h = jnp.dot(pre, l1w[...], preferred_element_type=jnp.float32) + l1b[...]
    h_out[...] = h

    @pl.when(pl.program_id(0) == 0)
    def _():
        s_out[...] = jnp.zeros_like(s_out)
        sq_out[...] = jnp.zeros_like(sq_out)

    s_out[...] += jnp.sum(h, axis=0, keepdims=True)
    sq_out[...] += jnp.sum(h * h, axis=0, keepdims=True)


def _tc_e1(ef, emsg, scg, p):
    nsteps = E // BE
    row = lambda i: (i, 0)
    fixed = lambda i: (0, 0)
    return pl.pallas_call(
        _tc_e1_body,
        grid=(nsteps,),
        in_specs=[
            pl.BlockSpec((BE, D), row),
            pl.BlockSpec((BE, D), row),
            pl.BlockSpec((BE, D), row),
            pl.BlockSpec((D, D), fixed),
            pl.BlockSpec((D, D), fixed),
            pl.BlockSpec((D, D), fixed),
            pl.BlockSpec((1, D), fixed),
            pl.BlockSpec((D, D), fixed),
            pl.BlockSpec((1, D), fixed),
        ],
        out_specs=[
            pl.BlockSpec((BE, D), row),
            pl.BlockSpec((1, D), fixed),
            pl.BlockSpec((1, D), fixed),
        ],
        out_shape=[
            jax.ShapeDtypeStruct((E, D), jnp.float32),
            jax.ShapeDtypeStruct((1, D), jnp.float32),
            jax.ShapeDtypeStruct((1, D), jnp.float32),
        ],
    )(ef, emsg, scg, p["eloop_w"], p["src_w"], p["dst_w"],
      p["ebias"].reshape(1, D), p["e_l1_w"], p["e_l1_b"].reshape(1, D))


def _tc_e2_body(h_ref, s_ref, sq_ref, g_ref, b_ref, l2w, l2b, out_ref):
    mu = s_ref[...] * (1.0 / E)
    var = sq_ref[...] * (1.0 / E) - mu * mu
    hn = (h_ref[...] - mu) * lax.rsqrt(var + 1e-5) * g_ref[...] + b_ref[...]
    out_ref[...] = (jnp.dot(jnp.maximum(hn, 0.0), l2w[...],
                            preferred_element_type=jnp.float32) + l2b[...])


def _tc_e2(h, s, sq, p):
    nsteps = E // BE
    row = lambda i: (i, 0)
    fixed = lambda i: (0, 0)
    return pl.pallas_call(
        _tc_e2_body,
        grid=(nsteps,),
        in_specs=[
            pl.BlockSpec((BE, D), row),
            pl.BlockSpec((1, D), fixed),
            pl.BlockSpec((1, D), fixed),
            pl.BlockSpec((1, D), fixed),
            pl.BlockSpec((1, D), fixed),
            pl.BlockSpec((D, D), fixed),
            pl.BlockSpec((1, D), fixed),
        ],
        out_specs=pl.BlockSpec((BE, D), row),
        out_shape=jax.ShapeDtypeStruct((E, D), jnp.float32),
    )(h, s, sq, p["e_bn_g"].reshape(1, D), p["e_bn_b"].reshape(1, D),
      p["e_l2_w"], p["e_l2_b"].reshape(1, D))


def _tc_node_body(nf, agg0, agg1, inw, nloopw, nb, l1w, l1b, g, b, l2w, l2b,
                  out_ref):
    x = nf[...]
    agg = jnp.concatenate([agg0[...], agg1[...]], axis=0)[:N]
    pre = (jnp.dot(x, nloopw[...], preferred_element_type=jnp.float32)
           - jnp.dot(agg, inw[...], preferred_element_type=jnp.float32)
           + nb[...])
    h = jnp.dot(pre, l1w[...], preferred_element_type=jnp.float32) + l1b[...]
    mu = jnp.mean(h, axis=0, keepdims=True)
    var = jnp.mean((h - mu) ** 2, axis=0, keepdims=True)
    hn = (h - mu) * lax.rsqrt(var + 1e-5) * g[...] + b[...]
    out_ref[...] = (jnp.dot(jnp.maximum(hn, 0.0), l2w[...],
                            preferred_element_type=jnp.float32) + l2b[...])


def _tc_node(nf, agg0, agg1, p):
    return pl.pallas_call(
        _tc_node_body,
        out_shape=jax.ShapeDtypeStruct((N, D), jnp.float32),
    )(nf, agg0, agg1, p["in_w"], p["nloop_w"], p["nbias"].reshape(1, D),
      p["n_l1_w"], p["n_l1_b"].reshape(1, D),
      p["n_bn_g"].reshape(1, D), p["n_bn_b"].reshape(1, D),
      p["n_l2_w"], p["n_l2_b"].reshape(1, D))


def _tc_lrp_body(gn, wf, lb, pool_ref, out_ref):
    ps = gn[...]
    mm = jnp.dot(ps, wf[...], preferred_element_type=jnp.float32) + lb[...]
    pid = pool_ref[0, 0, :]
    oh = (pid[:, None] == lax.broadcasted_iota(jnp.int32, (BP, G), 1)
          ).astype(jnp.float32)
    contrib = lax.dot_general(oh, mm, (((0,), (0,)), ((), ())),
                              preferred_element_type=jnp.float32)

    @pl.when(pl.program_id(0) == 0)
    def _():
        out_ref[...] = jnp.zeros_like(out_ref)

    out_ref[...] += contrib


def _tc_lrp(gn2, wflat, lrp_bias, pool3):
    nsteps = P // BP
    row = lambda i: (i, 0)
    fixed = lambda i: (0, 0)
    return pl.pallas_call(
        _tc_lrp_body,
        grid=(nsteps,),
        in_specs=[
            pl.BlockSpec((BP, L2 * D), row),
            pl.BlockSpec((L2 * D, D), fixed),
            pl.BlockSpec((1, D), fixed),
            pl.BlockSpec((1, 1, BP), lambda i: (i, 0, 0)),
        ],
        out_specs=pl.BlockSpec((G, D), fixed),
        out_shape=jax.ShapeDtypeStruct((G, D), jnp.float32),
    )(gn2, wflat, lrp_bias, pool3)


def kernel(node_feat, edge_feat, params, edge_index, node_perm_idx,
           edge_perm_idx, pool_idx):
    p = params
    src = edge_index[0]
    dst = edge_index[1]

    znd = jnp.zeros((NPAD4, D), jnp.float32)

    ones_blk = jnp.ones((KE, D), jnp.float32)
    agg0 = _sc_scatter_q0(edge_feat, dst, znd)
    agg1 = _sc_scatter_q1(edge_feat, dst, znd)
    deg0 = _sc_deg_q0(src, znd, ones_blk)
    deg1 = _sc_deg_q1(src, znd, ones_blk)
    nfd, nfs, stab = _tc_prep(node_feat, p["dst_w"], p["src_w"], deg0, deg1)
    emsg, scg = _sc_edge_gather(nfd, nfs, stab, src, dst)
    h, s, sq = _tc_e1(edge_feat, emsg, scg, p)
    edge_out = _tc_e2(h, s, sq, p)
    node_out = _tc_node(node_feat, agg0, agg1, p)
    psum = _sc_perm_gather(node_out, edge_out, node_perm_idx,
                           edge_perm_idx)
    ps2 = psum.reshape(P, L2 * D)
    wflat = jnp.transpose(p["lrp_w"], (2, 0, 1)).reshape(L2 * D, D)
    pool3 = pool_idx.reshape(P // BP, 1, BP)
    pooled = _tc_lrp(ps2, wflat, p["lrp_bias"].reshape(1, D), pool3)
    return pooled, edge_out
